# Initial kernel scaffold; baseline (speedup 1.0000x reference)
import jax
import jax.numpy as jnp
from jax.experimental import pallas as pl


def kernel(semantic_tokens, table, W1, b1, W2, b2, W3, b3):
    raise NotImplementedError("write your pallas kernel here")


# speedup vs baseline: 4.2557x; 4.2557x over previous
import jax
import jax.numpy as jnp
from jax import lax
from jax.experimental import pallas as pl
from jax.experimental.pallas import tpu as pltpu
from jax.experimental.pallas import tpu_sc as plsc

VOCAB = 100000
EMB = 512
B = 16384
L = 20

NC = 2
NS = 16
NW = NC * NS
BPW = B // NW
CB = 4
CHUNK = CB * L
CB_OUT = 2 * CB
N_OUT = BPW // CB_OUT
LANES = 16
NCOL = EMB // LANES


def _sc_pool_body(table_hbm, tok_hbm, out_hbm,
                  idx0, idx1, rows0, rows1, stage_v,
                  sem0, sem1):
    wid = lax.axis_index("s") * NC + lax.axis_index("c")
    idx_base = wid * (BPW * L)
    out_base = wid * BPW

    idx_bufs = [idx0, idx1]
    row_bufs = [rows0, rows1]
    sems = [sem0, sem1]

    def gather_start(g, buf):
        pltpu.sync_copy(tok_hbm.at[pl.ds(idx_base + g * CHUNK, CHUNK)],
                        idx_bufs[buf])
        pltpu.async_copy(table_hbm.at[idx_bufs[buf]], row_bufs[buf],
                         sems[buf])

    def gather_wait(buf):
        pltpu.make_async_copy(table_hbm.at[idx_bufs[buf]], row_bufs[buf],
                              sems[buf]).wait()

    def pool_chunk(rows, stage_base):
        def col_body(c, carry):
            sl = pl.ds(c * LANES, LANES)
            for b in range(CB):
                s0 = rows[b * L + 0, sl]
                s1 = rows[b * L + 1, sl]
                for r in range(2, L, 2):
                    s0 = s0 + rows[b * L + r, sl]
                    s1 = s1 + rows[b * L + r + 1, sl]
                stage_v[stage_base + b, sl] = s0 + s1
            return carry

        lax.fori_loop(0, NCOL, col_body, 0)

    gather_start(0, 0)
    gather_start(1, 1)

    def iter_body(i, carry):
        gather_wait(0)
        pool_chunk(rows0, 0)

        @pl.when(i < N_OUT - 1)
        def _():
            gather_start(2 * i + 2, 0)

        gather_wait(1)
        pool_chunk(rows1, CB)

        @pl.when(i < N_OUT - 1)
        def _():
            gather_start(2 * i + 3, 1)

        pltpu.sync_copy(stage_v,
                        out_hbm.at[pl.ds(out_base + i * CB_OUT, CB_OUT)])
        return carry

    lax.fori_loop(0, N_OUT, iter_body, 0)


def _sc_pool(table, tokens_flat):
    pool = pl.kernel(
        _sc_pool_body,
        out_type=jax.ShapeDtypeStruct((B, EMB), jnp.float32),
        mesh=plsc.VectorSubcoreMesh(core_axis_name="c", subcore_axis_name="s"),
        scratch_types=[
            pltpu.VMEM((CHUNK,), jnp.int32),
            pltpu.VMEM((CHUNK,), jnp.int32),
            pltpu.VMEM((CHUNK, EMB), jnp.float32),
            pltpu.VMEM((CHUNK, EMB), jnp.float32),
            pltpu.VMEM((CB_OUT, EMB), jnp.float32),
            pltpu.SemaphoreType.DMA,
            pltpu.SemaphoreType.DMA,
        ],
    )
    return pool(table, tokens_flat)


def _mlp_body(x_ref, w1_ref, b1_ref, w2_ref, b2_ref, w3_ref, b3_ref, o_ref):
    dn = (((1,), (1,)), ((), ()))
    x = x_ref[...] * (1.0 / L)
    h = lax.dot_general(x, w1_ref[...], dn, preferred_element_type=jnp.float32)
    h = jnp.maximum(h + b1_ref[...], 0.0)
    h = lax.dot_general(h, w2_ref[...], dn, preferred_element_type=jnp.float32)
    h = jnp.maximum(h + b2_ref[...], 0.0)
    o = lax.dot_general(h, w3_ref[...], dn, preferred_element_type=jnp.float32)
    o_ref[...] = o + b3_ref[...]


def _mlp(pooled, W1, b1, W2, b2, W3, b3):
    TB = 1024
    return pl.pallas_call(
        _mlp_body,
        grid=(B // TB,),
        in_specs=[
            pl.BlockSpec((TB, EMB), lambda i: (i, 0)),
            pl.BlockSpec((256, EMB), lambda i: (0, 0)),
            pl.BlockSpec((1, 256), lambda i: (0, 0)),
            pl.BlockSpec((128, 256), lambda i: (0, 0)),
            pl.BlockSpec((1, 128), lambda i: (0, 0)),
            pl.BlockSpec((64, 128), lambda i: (0, 0)),
            pl.BlockSpec((1, 64), lambda i: (0, 0)),
        ],
        out_specs=pl.BlockSpec((TB, 64), lambda i: (i, 0)),
        out_shape=jax.ShapeDtypeStruct((B, 64), jnp.float32),
    )(pooled, W1, b1.reshape(1, -1), W2, b2.reshape(1, -1), W3,
      b3.reshape(1, -1))


def kernel(semantic_tokens, table, W1, b1, W2, b2, W3, b3):
    tokens_flat = semantic_tokens.reshape(-1)
    pooled_sum = _sc_pool(table, tokens_flat)
    return _mlp(pooled_sum, W1, b1, W2, b2, W3, b3)
